# Initial kernel scaffold; baseline (speedup 1.0000x reference)
import jax
import jax.numpy as jnp
from jax.experimental import pallas as pl


def kernel(input_tensor, Wg, bg, We, be):
    raise NotImplementedError("write your pallas kernel here")


# speedup vs baseline: 2.1443x; 2.1443x over previous
import functools

import jax
import jax.numpy as jnp
from jax import lax
from jax.experimental import pallas as pl
from jax.experimental.pallas import tpu as pltpu

B, S, D, E, K = 4, 2048, 768, 8, 2
T = B * S
TM = 256
EPAD = 128


def _fused_moe_kernel(x_ref, wg_ref, bg_ref, we_ref, be_ref, out_ref):
    x = x_ref[...]
    logits = jnp.dot(x, wg_ref[...], preferred_element_type=jnp.float32)
    logits = logits + bg_ref[0][None, :]
    mx = jnp.max(logits, axis=-1, keepdims=True)
    ex = jnp.exp(logits - mx)
    p = ex / jnp.sum(ex, axis=-1, keepdims=True)
    ii = lax.broadcasted_iota(jnp.int32, (TM, EPAD), 1)
    m0 = jnp.max(p, axis=-1, keepdims=True)
    e0 = jnp.min(jnp.where(p == m0, ii, EPAD), axis=-1, keepdims=True)
    sel0 = ii == e0
    p1 = jnp.where(sel0, -1.0, p)
    m1 = jnp.max(p1, axis=-1, keepdims=True)
    e1 = jnp.min(jnp.where(p1 == m1, ii, EPAD), axis=-1, keepdims=True)
    comb = jnp.where(sel0, m0, 0.0) + jnp.where(ii == e1, m1, 0.0)

    xb = x.astype(jnp.bfloat16)
    acc = jnp.zeros((TM, D), jnp.float32)
    for e in range(E):
        y = jnp.dot(xb, we_ref[e], preferred_element_type=jnp.float32)
        y = y + be_ref[e][None, :]
        acc = acc + y * comb[:, e:e + 1]
    out_ref[...] = acc


@functools.partial(jax.jit, static_argnames=())
def kernel(input_tensor, Wg, bg, We, be):
    x = input_tensor.reshape(T, D)
    wg = jnp.pad(Wg, ((0, 0), (0, EPAD - E)))
    bgp = jnp.pad(bg, (0, EPAD - E), constant_values=-1e30).reshape(1, EPAD)
    we_bf = We.astype(jnp.bfloat16)

    out = pl.pallas_call(
        _fused_moe_kernel,
        grid=(T // TM,),
        in_specs=[
            pl.BlockSpec((TM, D), lambda m: (m, 0)),
            pl.BlockSpec((D, EPAD), lambda m: (0, 0)),
            pl.BlockSpec((1, EPAD), lambda m: (0, 0)),
            pl.BlockSpec((E, D, D), lambda m: (0, 0, 0)),
            pl.BlockSpec((E, D), lambda m: (0, 0)),
        ],
        out_specs=pl.BlockSpec((TM, D), lambda m: (m, 0)),
        out_shape=jax.ShapeDtypeStruct((T, D), jnp.float32),
    )(x, wg, bgp, we_bf, be)
    return out.reshape(B, S, D)
